# Initial kernel scaffold; baseline (speedup 1.0000x reference)
import jax
import jax.numpy as jnp
from jax.experimental import pallas as pl


def kernel(coordinates, distsq, features, n_idxs, W0, b0, W1, b1, W2, b2):
    raise NotImplementedError("write your pallas kernel here")


# speedup vs baseline: 1.6350x; 1.6350x over previous
import functools

import jax
import jax.numpy as jnp
from jax import lax
from jax.experimental import pallas as pl
from jax.experimental.pallas import tpu as pltpu
from jax.experimental.pallas import tpu_sc as plsc

V = 10000
K = 32
F = 128
C = 3
H = 64
NW = 32
VPW = 320
VPAD = NW * VPW
CHUNK = 4
ROWS = CHUNK * K
VB = 16


def _sc_gather_body(nidx_hbm, feat_hbm, coordt_hbm, outf_hbm, outc_hbm,
                    idx_v, fbuf, coordt_v, ncbuf, sem_f):
    wid = lax.axis_index("s") * 2 + lax.axis_index("c")
    pltpu.sync_copy(coordt_hbm, coordt_v)
    vbase = wid * VPW

    def body(i, carry):
        vb = vbase + i * CHUNK
        eb = vb * K
        pltpu.sync_copy(nidx_hbm.at[pl.ds(eb, ROWS)], idx_v)
        pltpu.async_copy(feat_hbm.at[idx_v], fbuf, sem_f).wait()
        for vi in range(CHUNK):
            for h in range(K // 16):
                idx16 = idx_v[pl.ds(vi * K + h * 16, 16)]
                for c in range(C):
                    vals = plsc.load_gather(coordt_v, [idx16 + c * VPAD])
                    ncbuf[vi, c, pl.ds(h * 16, 16)] = vals
        pltpu.sync_copy(fbuf, outf_hbm.at[pl.ds(eb, ROWS)])
        pltpu.sync_copy(ncbuf, outc_hbm.at[pl.ds(vb, CHUNK)])
        return carry

    lax.fori_loop(0, VPW // CHUNK, body, 0)


def _make_sc_gather():
    mesh = plsc.VectorSubcoreMesh(core_axis_name="c", subcore_axis_name="s",
                                  num_cores=2, num_subcores=16)
    return pl.kernel(
        _sc_gather_body,
        mesh=mesh,
        out_type=[
            jax.ShapeDtypeStruct((VPAD * K, F), jnp.float32),
            jax.ShapeDtypeStruct((VPAD, 8, K), jnp.float32),
        ],
        scratch_types=[
            pltpu.VMEM((ROWS,), jnp.int32),
            pltpu.VMEM((ROWS, F), jnp.float32),
            pltpu.VMEM((4 * VPAD,), jnp.float32),
            pltpu.VMEM((CHUNK, 8, K), jnp.float32),
            pltpu.SemaphoreType.DMA,
        ],
        compiler_params=pltpu.CompilerParams(needs_layout_passes=False),
    )


def _tc_body(distsq_ref, nct_ref, nf_ref, w0_ref, b0_ref, w1_ref, b1_ref,
             w2_ref, b2_ref, out_ref):
    e = jnp.exp(-10.0 * distsq_ref[...])[:, None, :]
    nct = nct_ref[...]
    rows = []
    for c in range(C):
        for d in range(C):
            rows.append(nct[:, c:c + 1, :] * nct[:, d:d + 1, :] * e)
    for c in range(C):
        rows.append(nct[:, c:c + 1, :] * e)
    rows.append(e)
    rows.append(jnp.zeros((VB, 3, K), jnp.float32))
    qt = jnp.concatenate(rows, axis=1)
    nf = nf_ref[...].reshape(VB, K, F)
    m = lax.dot_general(qt, nf, (((2,), (1,)), ((0,), (0,))),
                        preferred_element_type=jnp.float32)
    recip = 1.0 / (m[:, 12:13, :] + 1e-4)
    mean = m[:, 9:12, :] * recip
    exx = m[:, 0:9, :] * recip
    crows = []
    for c in range(C):
        for d in range(C):
            j = 3 * c + d
            crows.append(exx[:, j:j + 1, :] - mean[:, c:c + 1, :] * mean[:, d:d + 1, :])
    crows.append(jnp.zeros((VB, 7, F), jnp.float32))
    cov = jnp.concatenate(crows, axis=1)
    covt = jnp.swapaxes(cov, 1, 2).reshape(VB * F, 16)
    x = covt @ w0_ref[...] + b0_ref[...]
    x = jnp.where(x > 0, x, jnp.exp(x) - 1.0)
    x = x @ w1_ref[...] + b1_ref[...]
    x = jnp.where(x > 0, x, jnp.exp(x) - 1.0)
    out_ref[...] = x @ w2_ref[...] + b2_ref[...]


def _make_tc_call():
    grid = (V // VB,)
    return pl.pallas_call(
        _tc_body,
        grid=grid,
        in_specs=[
            pl.BlockSpec((VB, K), lambda i: (i, 0)),
            pl.BlockSpec((VB, 8, K), lambda i: (i, 0, 0)),
            pl.BlockSpec((VB * K, F), lambda i: (i, 0)),
            pl.BlockSpec((16, H), lambda i: (0, 0)),
            pl.BlockSpec((1, H), lambda i: (0, 0)),
            pl.BlockSpec((H, H), lambda i: (0, 0)),
            pl.BlockSpec((1, H), lambda i: (0, 0)),
            pl.BlockSpec((H, 16), lambda i: (0, 0)),
            pl.BlockSpec((1, 16), lambda i: (0, 0)),
        ],
        out_specs=pl.BlockSpec((VB * F, 16), lambda i: (i, 0)),
        out_shape=jax.ShapeDtypeStruct((V * F, 16), jnp.float32),
    )


_tc_call = _make_tc_call()


def kernel(coordinates, distsq, features, n_idxs, W0, b0, W1, b1, W2, b2):
    nidx = n_idxs.astype(jnp.int32)
    nidx_pad = jnp.zeros((VPAD, K), jnp.int32).at[:V].set(nidx).reshape(VPAD * K)
    coordt = jnp.zeros((4, VPAD), jnp.float32).at[:C, :V].set(
        coordinates.T).reshape(4 * VPAD)
    nf_g, nct = _make_sc_gather()(nidx_pad, features, coordt)
    w0p = jnp.zeros((16, H), jnp.float32).at[:C * C].set(W0)
    w2p = jnp.zeros((H, 16), jnp.float32).at[:, :C * C].set(W2)
    b2p = jnp.zeros((1, 16), jnp.float32).at[0, :C * C].set(b2)
    out = _tc_call(distsq, nct, nf_g, w0p, b0.reshape(1, H), W1,
                   b1.reshape(1, H), w2p, b2p)
    return out[:, :C * C].reshape(V, F * C * C)


# speedup vs baseline: 2.1682x; 1.3262x over previous
import jax
import jax.numpy as jnp
from jax import lax
from jax.experimental import pallas as pl
from jax.experimental.pallas import tpu as pltpu
from jax.experimental.pallas import tpu_sc as plsc

V = 10000
K = 32
F = 128
C = 3
H = 64
HE = 72
NW = 32
VPW = 320
VPAD = NW * VPW
CHUNK = 4
ROWS = CHUNK * K
NCH = VPW // CHUNK
VB = 80


def _sc_gather_body(nidx_hbm, dsq_hbm, feat_hbm, coordt_hbm, outf_hbm,
                    outq_hbm, idx_all, dsq_all, coordt_v, fbuf0, fbuf1,
                    qbuf0, qbuf1, sem_g0, sem_g1, sem_w0, sem_w1, sem_q0,
                    sem_q1):
    wid = lax.axis_index("s") * 2 + lax.axis_index("c")
    pltpu.sync_copy(coordt_hbm, coordt_v)
    pltpu.sync_copy(nidx_hbm.at[pl.ds(wid * NCH, NCH)], idx_all)
    pltpu.sync_copy(dsq_hbm.at[pl.ds(wid * NCH, NCH)], dsq_all)
    vbase = wid * VPW
    fbufs = (fbuf0, fbuf1)
    qbufs = (qbuf0, qbuf1)
    sems_g = (sem_g0, sem_g1)
    sems_w = (sem_w0, sem_w1)
    sems_q = (sem_q0, sem_q1)

    zero16 = jnp.zeros((16,), jnp.float32)
    for qb in qbufs:
        for vi in range(CHUNK):
            for r in (13, 14, 15):
                for h in range(K // 16):
                    qb[vi, r, pl.ds(h * 16, 16)] = zero16

    def _qrows(j, qbuf):
        for vi in range(CHUNK):
            for h in range(K // 16):
                sl = pl.ds(vi * K + h * 16, 16)
                lsl = pl.ds(h * 16, 16)
                idx16 = idx_all[j, sl]
                e16 = jnp.exp(dsq_all[j, sl] * -10.0)
                cc = [plsc.load_gather(coordt_v, [idx16 + c * VPAD])
                      for c in range(C)]
                for c in range(C):
                    for d in range(c, C):
                        p = cc[c] * cc[d] * e16
                        qbuf[vi, 3 * c + d, lsl] = p
                        if d != c:
                            qbuf[vi, 3 * d + c, lsl] = p
                    qbuf[vi, 9 + c, lsl] = cc[c] * e16
                qbuf[vi, 12, lsl] = e16

    def _wb_descs(j, b):
        vb = vbase + j * CHUNK
        return (
            pltpu.make_async_copy(fbufs[b], outf_hbm.at[pl.ds(vb * K, ROWS)],
                                  sems_w[b]),
            pltpu.make_async_copy(qbufs[b], outq_hbm.at[pl.ds(vb, CHUNK)],
                                  sems_q[b]),
        )

    def _gather_desc(j, b):
        return pltpu.make_async_copy(feat_hbm.at[idx_all.at[j]], fbufs[b],
                                     sems_g[b])

    def body(p, carry):
        for b in range(2):
            j = 2 * p + b

            @pl.when(p > 0)
            def _():
                wf, wq = _wb_descs(j, b)
                wf.wait()
                wq.wait()

            _gather_desc(j, b).start()
        for b in range(2):
            j = 2 * p + b
            _qrows(j, qbufs[b])
            _gather_desc(j, b).wait()
            wf, wq = _wb_descs(j, b)
            wf.start()
            wq.start()
        return carry

    lax.fori_loop(0, NCH // 2, body, 0)
    for b in range(2):
        wf, wq = _wb_descs(NCH - 2 + b, b)
        wf.wait()
        wq.wait()


def _make_sc_gather():
    mesh = plsc.VectorSubcoreMesh(core_axis_name="c", subcore_axis_name="s",
                                  num_cores=2, num_subcores=16)
    return pl.kernel(
        _sc_gather_body,
        mesh=mesh,
        out_type=[
            jax.ShapeDtypeStruct((VPAD * K, F), jnp.float32),
            jax.ShapeDtypeStruct((VPAD, 16, K), jnp.float32),
        ],
        scratch_types=[
            pltpu.VMEM((NCH, ROWS), jnp.int32),
            pltpu.VMEM((NCH, ROWS), jnp.float32),
            pltpu.VMEM((C * VPAD,), jnp.float32),
            pltpu.VMEM((ROWS, F), jnp.float32),
            pltpu.VMEM((ROWS, F), jnp.float32),
            pltpu.VMEM((CHUNK, 16, K), jnp.float32),
            pltpu.VMEM((CHUNK, 16, K), jnp.float32),
            pltpu.SemaphoreType.DMA,
            pltpu.SemaphoreType.DMA,
            pltpu.SemaphoreType.DMA,
            pltpu.SemaphoreType.DMA,
            pltpu.SemaphoreType.DMA,
            pltpu.SemaphoreType.DMA,
        ],
        compiler_params=pltpu.CompilerParams(needs_layout_passes=False),
    )


def _prep_weights(W0, b0, W1, b1, W2, b2):
    w0e = jnp.zeros((HE, 16), jnp.float32)
    w0e = w0e.at[:H, :C * C].set(W0.T).at[:H, 9].set(b0).at[H, 9].set(1.0)
    w1e = jnp.zeros((HE, HE), jnp.float32)
    w1e = w1e.at[:H, :H].set(W1.T).at[:H, H].set(b1).at[H, H].set(1.0)
    w2e = jnp.zeros((16, HE), jnp.float32)
    w2e = w2e.at[:C * C, :H].set(W2.T).at[:C * C, H].set(b2)
    return w0e, w1e, w2e


def _tc_body(qt_ref, nf_ref, w0_ref, w1_ref, w2_ref, out_ref):
    qt = qt_ref[...]
    nf = nf_ref[...].reshape(VB, K, F)
    m = lax.dot_general(qt, nf, (((2,), (1,)), ((0,), (0,))),
                        preferred_element_type=jnp.float32)
    m2 = jnp.concatenate([m[v] for v in range(VB)], axis=1)
    recip = 1.0 / (m2[12:13, :] + 1e-4)
    mean = m2[9:12, :] * recip
    exx = m2[0:9, :] * recip
    crows = [exx[3 * c + d:3 * c + d + 1] - mean[c:c + 1] * mean[d:d + 1]
             for c in range(C) for d in range(C)]
    crows.append(jnp.ones((1, VB * F), jnp.float32))
    crows.append(jnp.zeros((6, VB * F), jnp.float32))
    cov2 = jnp.concatenate(crows, axis=0)
    x = lax.dot_general(w0_ref[...], cov2, (((1,), (0,)), ((), ())),
                        preferred_element_type=jnp.float32)
    x = jnp.where(x > 0, x, jnp.exp(x) - 1.0)
    x = lax.dot_general(w1_ref[...], x, (((1,), (0,)), ((), ())),
                        preferred_element_type=jnp.float32)
    x = jnp.where(x > 0, x, jnp.exp(x) - 1.0)
    out_ref[...] = lax.dot_general(w2_ref[...], x, (((1,), (0,)), ((), ())),
                                   preferred_element_type=jnp.float32)


def _make_tc_call():
    return pl.pallas_call(
        _tc_body,
        grid=(V // VB,),
        in_specs=[
            pl.BlockSpec((VB, 16, K), lambda i: (i, 0, 0)),
            pl.BlockSpec((VB * K, F), lambda i: (i, 0)),
            pl.BlockSpec((HE, 16), lambda i: (0, 0)),
            pl.BlockSpec((HE, HE), lambda i: (0, 0)),
            pl.BlockSpec((16, HE), lambda i: (0, 0)),
        ],
        out_specs=pl.BlockSpec((16, VB * F), lambda i: (0, i)),
        out_shape=jax.ShapeDtypeStruct((16, V * F), jnp.float32),
    )


_tc_call = _make_tc_call()


def kernel(coordinates, distsq, features, n_idxs, W0, b0, W1, b1, W2, b2):
    nidx = n_idxs.astype(jnp.int32)
    nidx_pad = jnp.zeros((VPAD, K), jnp.int32).at[:V].set(nidx)
    nidx_pad = nidx_pad.reshape(VPAD * K // ROWS, ROWS)
    dsq_pad = jnp.zeros((VPAD, K), jnp.float32).at[:V].set(distsq)
    dsq_pad = dsq_pad.reshape(VPAD * K // ROWS, ROWS)
    coordt = jnp.zeros((C, VPAD), jnp.float32).at[:, :V].set(
        coordinates.T).reshape(C * VPAD)
    nf_g, qt_g = _make_sc_gather()(nidx_pad, dsq_pad, features, coordt)
    w0e, w1e, w2e = _prep_weights(W0, b0, W1, b1, W2, b2)
    out = _tc_call(qt_g, nf_g, w0e, w1e, w2e)
    return out[:C * C].T.reshape(V, F * C * C)


# speedup vs baseline: 2.1938x; 1.0118x over previous
import jax
import jax.numpy as jnp
from jax import lax
from jax.experimental import pallas as pl
from jax.experimental.pallas import tpu as pltpu
from jax.experimental.pallas import tpu_sc as plsc

V = 10000
K = 32
F = 128
C = 3
H = 64
HE = 72
NW = 32
VPW = 320
VPAD = NW * VPW
CHUNK = 4
ROWS = CHUNK * K
NCH = VPW // CHUNK
NBUF = 3
VB = 80


def _sc_gather_body(nidx_hbm, dsq_hbm, feat_hbm, coordt_hbm, outf_hbm,
                    outq_hbm, idx_all, dsq_all, coordt_v, *bufs_and_sems):
    fbufs = bufs_and_sems[0:NBUF]
    qbufs = bufs_and_sems[NBUF:2 * NBUF]
    sems_g = bufs_and_sems[2 * NBUF:3 * NBUF]
    sems_w = bufs_and_sems[3 * NBUF:4 * NBUF]
    sems_q = bufs_and_sems[4 * NBUF:5 * NBUF]
    wid = lax.axis_index("s") * 2 + lax.axis_index("c")
    pltpu.sync_copy(coordt_hbm, coordt_v)
    pltpu.sync_copy(nidx_hbm.at[pl.ds(wid * NCH, NCH)], idx_all)
    pltpu.sync_copy(dsq_hbm.at[pl.ds(wid * NCH, NCH)], dsq_all)
    vbase = wid * VPW

    zero16 = jnp.zeros((16,), jnp.float32)
    for qb in qbufs:
        for vi in range(CHUNK):
            for r in (13, 14, 15):
                for h in range(K // 16):
                    qb[vi, r, pl.ds(h * 16, 16)] = zero16

    def _qrows(j, qbuf):
        for vi in range(CHUNK):
            for h in range(K // 16):
                sl = pl.ds(vi * K + h * 16, 16)
                lsl = pl.ds(h * 16, 16)
                idx16 = idx_all[j, sl]
                e16 = jnp.exp(dsq_all[j, sl] * -10.0)
                cc = [plsc.load_gather(coordt_v, [idx16 + c * V])
                      for c in range(C)]
                for c in range(C):
                    for d in range(c, C):
                        p = cc[c] * cc[d] * e16
                        qbuf[vi, 3 * c + d, lsl] = p
                        if d != c:
                            qbuf[vi, 3 * d + c, lsl] = p
                    qbuf[vi, 9 + c, lsl] = cc[c] * e16
                qbuf[vi, 12, lsl] = e16

    def _wbf_desc(j, b):
        vb = vbase + j * CHUNK
        return pltpu.make_async_copy(fbufs[b],
                                     outf_hbm.at[pl.ds(vb * K, ROWS)],
                                     sems_w[b])

    def _wbq_desc(j, qb):
        vb = vbase + j * CHUNK
        return pltpu.make_async_copy(qbufs[qb],
                                     outq_hbm.at[pl.ds(vb, CHUNK)],
                                     sems_q[qb])

    def _gather_desc(j, b):
        return pltpu.make_async_copy(feat_hbm.at[idx_all.at[j]], fbufs[b],
                                     sems_g[b])

    def body(p, carry):
        for b in range(NBUF):
            j = NBUF * p + b

            @pl.when(p > 0)
            def _():
                _wbf_desc(j, b).wait()
                _wbq_desc(j, b).wait()

            _gather_desc(j, b).start()
        for b in range(NBUF):
            j = NBUF * p + b
            _qrows(j, qbufs[b])
            _gather_desc(j, b).wait()
            _wbf_desc(j, b).start()
            _wbq_desc(j, b).start()
        return carry

    nfull = NCH // NBUF
    lax.fori_loop(0, nfull, body, 0)
    for j in range(nfull * NBUF, NCH):
        b = j % NBUF
        _wbf_desc(j, b).wait()
        _wbq_desc(j, b).wait()
        _gather_desc(j, b).start()
        _qrows(j, qbufs[b])
        _gather_desc(j, b).wait()
        _wbf_desc(j, b).start()
        _wbq_desc(j, b).start()
    for j in range(NCH - NBUF, NCH):
        b = j % NBUF
        _wbf_desc(j, b).wait()
        _wbq_desc(j, b).wait()


def _make_sc_gather():
    mesh = plsc.VectorSubcoreMesh(core_axis_name="c", subcore_axis_name="s",
                                  num_cores=2, num_subcores=16)
    return pl.kernel(
        _sc_gather_body,
        mesh=mesh,
        out_type=[
            jax.ShapeDtypeStruct((VPAD * K, F), jnp.float32),
            jax.ShapeDtypeStruct((VPAD, 16, K), jnp.float32),
        ],
        scratch_types=(
            [
                pltpu.VMEM((NCH, ROWS), jnp.int32),
                pltpu.VMEM((NCH, ROWS), jnp.float32),
                pltpu.VMEM((C * V,), jnp.float32),
            ]
            + [pltpu.VMEM((ROWS, F), jnp.float32) for _ in range(NBUF)]
            + [pltpu.VMEM((CHUNK, 16, K), jnp.float32) for _ in range(NBUF)]
            + [pltpu.SemaphoreType.DMA for _ in range(3 * NBUF)]
        ),
        compiler_params=pltpu.CompilerParams(needs_layout_passes=False),
    )


def _prep_weights(W0, b0, W1, b1, W2, b2):
    w0e = jnp.zeros((HE, 16), jnp.float32)
    w0e = w0e.at[:H, :C * C].set(W0.T).at[:H, 9].set(b0).at[H, 9].set(1.0)
    w1e = jnp.zeros((HE, HE), jnp.float32)
    w1e = w1e.at[:H, :H].set(W1.T).at[:H, H].set(b1).at[H, H].set(1.0)
    w2e = jnp.zeros((16, HE), jnp.float32)
    w2e = w2e.at[:C * C, :H].set(W2.T).at[:C * C, H].set(b2)
    return w0e, w1e, w2e


def _tc_body(qt_ref, nf_ref, w0_ref, w1_ref, w2_ref, out_ref):
    qt = qt_ref[...]
    nf = nf_ref[...].reshape(VB, K, F)
    m = lax.dot_general(qt, nf, (((2,), (1,)), ((0,), (0,))),
                        preferred_element_type=jnp.float32)
    m2 = jnp.concatenate([m[v] for v in range(VB)], axis=1)
    recip = 1.0 / (m2[12:13, :] + 1e-4)
    mean = m2[9:12, :] * recip
    exx = m2[0:9, :] * recip
    crows = [exx[3 * c + d:3 * c + d + 1] - mean[c:c + 1] * mean[d:d + 1]
             for c in range(C) for d in range(C)]
    crows.append(jnp.ones((1, VB * F), jnp.float32))
    crows.append(jnp.zeros((6, VB * F), jnp.float32))
    cov2 = jnp.concatenate(crows, axis=0)
    x = lax.dot_general(w0_ref[...], cov2, (((1,), (0,)), ((), ())),
                        preferred_element_type=jnp.float32)
    x = jnp.where(x > 0, x, jnp.exp(x) - 1.0)
    x = lax.dot_general(w1_ref[...], x, (((1,), (0,)), ((), ())),
                        preferred_element_type=jnp.float32)
    x = jnp.where(x > 0, x, jnp.exp(x) - 1.0)
    out_ref[...] = lax.dot_general(w2_ref[...], x, (((1,), (0,)), ((), ())),
                                   preferred_element_type=jnp.float32)


def _make_tc_call():
    return pl.pallas_call(
        _tc_body,
        grid=(V // VB,),
        in_specs=[
            pl.BlockSpec((VB, 16, K), lambda i: (i, 0, 0)),
            pl.BlockSpec((VB * K, F), lambda i: (i, 0)),
            pl.BlockSpec((HE, 16), lambda i: (0, 0)),
            pl.BlockSpec((HE, HE), lambda i: (0, 0)),
            pl.BlockSpec((16, HE), lambda i: (0, 0)),
        ],
        out_specs=pl.BlockSpec((16, VB * F), lambda i: (0, i)),
        out_shape=jax.ShapeDtypeStruct((16, V * F), jnp.float32),
    )


_tc_call = _make_tc_call()


def kernel(coordinates, distsq, features, n_idxs, W0, b0, W1, b1, W2, b2):
    nidx = n_idxs.astype(jnp.int32)
    nidx_pad = jnp.zeros((VPAD, K), jnp.int32).at[:V].set(nidx)
    nidx_pad = nidx_pad.reshape(VPAD * K // ROWS, ROWS)
    dsq_pad = jnp.zeros((VPAD, K), jnp.float32).at[:V].set(distsq)
    dsq_pad = dsq_pad.reshape(VPAD * K // ROWS, ROWS)
    coordt = coordinates.T.reshape(C * V)
    nf_g, qt_g = _make_sc_gather()(nidx_pad, dsq_pad, features, coordt)
    w0e, w1e, w2e = _prep_weights(W0, b0, W1, b1, W2, b2)
    out = _tc_call(qt_g, nf_g, w0e, w1e, w2e)
    return out[:C * C].T.reshape(V, F * C * C)


# speedup vs baseline: 2.2522x; 1.0266x over previous
import jax
import jax.numpy as jnp
from jax import lax
from jax.experimental import pallas as pl
from jax.experimental.pallas import tpu as pltpu
from jax.experimental.pallas import tpu_sc as plsc

V = 10000
K = 32
F = 128
C = 3
H = 64
HE = 72
NW = 32
VPW = 320
VPAD = NW * VPW
CHUNK = 4
ROWS = CHUNK * K
NCH = VPW // CHUNK
NBUF = 3
VB = 80


def _sc_gather_body(nidx_hbm, dsq_hbm, feat_hbm, coordt_hbm, outf_hbm,
                    outq_hbm, idx_all, dsq_all, coordt_v, *bufs_and_sems):
    fbufs = bufs_and_sems[0:NBUF]
    qbufs = bufs_and_sems[NBUF:2 * NBUF]
    sems_g = bufs_and_sems[2 * NBUF:3 * NBUF]
    sems_w = bufs_and_sems[3 * NBUF:4 * NBUF]
    sems_q = bufs_and_sems[4 * NBUF:5 * NBUF]
    wid = lax.axis_index("s") * 2 + lax.axis_index("c")
    pltpu.sync_copy(coordt_hbm, coordt_v)
    pltpu.sync_copy(nidx_hbm.at[pl.ds(wid * NCH, NCH)], idx_all)
    pltpu.sync_copy(dsq_hbm.at[pl.ds(wid * NCH, NCH)], dsq_all)
    vbase = wid * VPW

    zero16 = jnp.zeros((16,), jnp.float32)
    for qb in qbufs:
        for vi in range(CHUNK):
            for r in (13, 14, 15):
                for h in range(K // 16):
                    qb[vi, r, pl.ds(h * 16, 16)] = zero16

    def _qrows(j, qbuf):
        for vi in range(CHUNK):
            for h in range(K // 16):
                sl = pl.ds(vi * K + h * 16, 16)
                lsl = pl.ds(h * 16, 16)
                idx16 = idx_all[j, sl]
                e16 = jnp.exp(dsq_all[j, sl] * -10.0)
                cc = [plsc.load_gather(coordt_v, [idx16 + c * V])
                      for c in range(C)]
                for c in range(C):
                    for d in range(c, C):
                        p = cc[c] * cc[d] * e16
                        qbuf[vi, 3 * c + d, lsl] = p
                        if d != c:
                            qbuf[vi, 3 * d + c, lsl] = p
                    qbuf[vi, 9 + c, lsl] = cc[c] * e16
                qbuf[vi, 12, lsl] = e16

    def _wbf_desc(j, b):
        vb = vbase + j * CHUNK
        return pltpu.make_async_copy(fbufs[b],
                                     outf_hbm.at[pl.ds(vb * K, ROWS)],
                                     sems_w[b])

    def _wbq_desc(j, qb):
        vb = vbase + j * CHUNK
        return pltpu.make_async_copy(qbufs[qb],
                                     outq_hbm.at[pl.ds(vb, CHUNK)],
                                     sems_q[qb])

    def _gather_desc(j, b):
        return pltpu.make_async_copy(feat_hbm.at[idx_all.at[j]], fbufs[b],
                                     sems_g[b])

    def body(p, carry):
        for b in range(NBUF):
            j = NBUF * p + b

            @pl.when(p > 0)
            def _():
                _wbf_desc(j, b).wait()
                _wbq_desc(j, b).wait()

            _gather_desc(j, b).start()
        for b in range(NBUF):
            j = NBUF * p + b
            _qrows(j, qbufs[b])
            _gather_desc(j, b).wait()
            _wbf_desc(j, b).start()
            _wbq_desc(j, b).start()
        return carry

    nfull = NCH // NBUF
    lax.fori_loop(0, nfull, body, 0)
    for j in range(nfull * NBUF, NCH):
        b = j % NBUF
        _wbf_desc(j, b).wait()
        _wbq_desc(j, b).wait()
        _gather_desc(j, b).start()
        _qrows(j, qbufs[b])
        _gather_desc(j, b).wait()
        _wbf_desc(j, b).start()
        _wbq_desc(j, b).start()
    for j in range(NCH - NBUF, NCH):
        b = j % NBUF
        _wbf_desc(j, b).wait()
        _wbq_desc(j, b).wait()


def _make_sc_gather():
    mesh = plsc.VectorSubcoreMesh(core_axis_name="c", subcore_axis_name="s",
                                  num_cores=2, num_subcores=16)
    return pl.kernel(
        _sc_gather_body,
        mesh=mesh,
        out_type=[
            jax.ShapeDtypeStruct((VPAD * K, F), jnp.float32),
            jax.ShapeDtypeStruct((VPAD, 16, K), jnp.float32),
        ],
        scratch_types=(
            [
                pltpu.VMEM((NCH, ROWS), jnp.int32),
                pltpu.VMEM((NCH, ROWS), jnp.float32),
                pltpu.VMEM((C * V,), jnp.float32),
            ]
            + [pltpu.VMEM((ROWS, F), jnp.float32) for _ in range(NBUF)]
            + [pltpu.VMEM((CHUNK, 16, K), jnp.float32) for _ in range(NBUF)]
            + [pltpu.SemaphoreType.DMA for _ in range(3 * NBUF)]
        ),
        compiler_params=pltpu.CompilerParams(needs_layout_passes=False),
    )


def _prep_weights(W0, b0, W1, b1, W2, b2):
    w0e = jnp.zeros((HE, 16), jnp.float32)
    w0e = w0e.at[:H, :C * C].set(W0.T).at[:H, 9].set(b0).at[H, 9].set(1.0)
    w1e = jnp.zeros((HE, HE), jnp.float32)
    w1e = w1e.at[:H, :H].set(W1.T).at[:H, H].set(b1).at[H, H].set(1.0)
    w2e = jnp.zeros((HE, C * C), jnp.float32)
    w2e = w2e.at[:H].set(W2).at[H].set(b2)
    return w0e, w1e, w2e


def _tc_body(qt_ref, nf_ref, w0_ref, w1_ref, w2_ref, out_ref):
    qt = qt_ref[...]
    nf = nf_ref[...].reshape(VB, K, F)
    m = lax.dot_general(qt, nf, (((2,), (1,)), ((0,), (0,))),
                        preferred_element_type=jnp.float32)
    m2 = jnp.concatenate([m[v] for v in range(VB)], axis=1)
    recip = 1.0 / (m2[12:13, :] + 1e-4)
    mean = m2[9:12, :] * recip
    exx = m2[0:9, :] * recip
    crows = [exx[3 * c + d:3 * c + d + 1] - mean[c:c + 1] * mean[d:d + 1]
             for c in range(C) for d in range(C)]
    crows.append(jnp.ones((1, VB * F), jnp.float32))
    crows.append(jnp.zeros((6, VB * F), jnp.float32))
    cov2 = jnp.concatenate(crows, axis=0)
    x = lax.dot_general(w0_ref[...], cov2, (((1,), (0,)), ((), ())),
                        preferred_element_type=jnp.float32)
    x = jnp.where(x > 0, x, jnp.exp(x) - 1.0)
    x = lax.dot_general(w1_ref[...], x, (((1,), (0,)), ((), ())),
                        preferred_element_type=jnp.float32)
    x = jnp.where(x > 0, x, jnp.exp(x) - 1.0)
    out_ref[...] = lax.dot_general(x, w2_ref[...], (((0,), (0,)), ((), ())),
                                   preferred_element_type=jnp.float32)


def _make_tc_call():
    return pl.pallas_call(
        _tc_body,
        grid=(V // VB,),
        in_specs=[
            pl.BlockSpec((VB, 16, K), lambda i: (i, 0, 0)),
            pl.BlockSpec((VB * K, F), lambda i: (i, 0)),
            pl.BlockSpec((HE, 16), lambda i: (0, 0)),
            pl.BlockSpec((HE, HE), lambda i: (0, 0)),
            pl.BlockSpec((HE, C * C), lambda i: (0, 0)),
        ],
        out_specs=pl.BlockSpec((VB * F, C * C), lambda i: (i, 0)),
        out_shape=jax.ShapeDtypeStruct((V * F, C * C), jnp.float32),
    )


_tc_call = _make_tc_call()


def kernel(coordinates, distsq, features, n_idxs, W0, b0, W1, b1, W2, b2):
    nidx = n_idxs.astype(jnp.int32)
    nidx_pad = jnp.zeros((VPAD, K), jnp.int32).at[:V].set(nidx)
    nidx_pad = nidx_pad.reshape(VPAD * K // ROWS, ROWS)
    dsq_pad = jnp.zeros((VPAD, K), jnp.float32).at[:V].set(distsq)
    dsq_pad = dsq_pad.reshape(VPAD * K // ROWS, ROWS)
    coordt = coordinates.T.reshape(C * V)
    nf_g, qt_g = _make_sc_gather()(nidx_pad, dsq_pad, features, coordt)
    w0e, w1e, w2e = _prep_weights(W0, b0, W1, b1, W2, b2)
    out = _tc_call(qt_g, nf_g, w0e, w1e, w2e)
    return out.reshape(V, F * C * C)


# speedup vs baseline: 2.4477x; 1.0868x over previous
import jax
import jax.numpy as jnp
from jax import lax
from jax.experimental import pallas as pl
from jax.experimental.pallas import tpu as pltpu
from jax.experimental.pallas import tpu_sc as plsc

V = 10000
K = 32
F = 128
C = 3
H = 64
HE = 72
NW = 32
VPW = 320
VPAD = NW * VPW
CHUNK = 4
ROWS = CHUNK * K
NCH = VPW // CHUNK
NBUF = 3
VB = 80


def _sc_gather_body(nidx_hbm, dsq_hbm, feat_hbm, coordt_hbm, outf_hbm,
                    outq_hbm, idx_all, dsq_all, coordt_v, *bufs_and_sems):
    fbufs = bufs_and_sems[0:NBUF]
    qbufs = bufs_and_sems[NBUF:2 * NBUF]
    sems_g = bufs_and_sems[2 * NBUF:3 * NBUF]
    sems_w = bufs_and_sems[3 * NBUF:4 * NBUF]
    sems_q = bufs_and_sems[4 * NBUF:5 * NBUF]
    wid = lax.axis_index("s") * 2 + lax.axis_index("c")
    pltpu.sync_copy(coordt_hbm, coordt_v)
    pltpu.sync_copy(nidx_hbm.at[pl.ds(wid * NCH, NCH)], idx_all)
    pltpu.sync_copy(dsq_hbm.at[pl.ds(wid * NCH, NCH)], dsq_all)
    vbase = wid * VPW

    zero16 = jnp.zeros((16,), jnp.float32)
    for qb in qbufs:
        for vi in range(CHUNK):
            for r in (13, 14, 15):
                for h in range(K // 16):
                    qb[vi, r, pl.ds(h * 16, 16)] = zero16

    def _qrows(j, qbuf):
        for vi in range(CHUNK):
            for h in range(K // 16):
                sl = pl.ds(vi * K + h * 16, 16)
                lsl = pl.ds(h * 16, 16)
                idx16 = idx_all[j, sl]
                e16 = jnp.exp(dsq_all[j, sl] * -10.0)
                cc = [plsc.load_gather(coordt_v, [idx16 + c * V])
                      for c in range(C)]
                for c in range(C):
                    for d in range(c, C):
                        p = cc[c] * cc[d] * e16
                        qbuf[vi, 3 * c + d, lsl] = p
                        if d != c:
                            qbuf[vi, 3 * d + c, lsl] = p
                    qbuf[vi, 9 + c, lsl] = cc[c] * e16
                qbuf[vi, 12, lsl] = e16

    def _wbf_desc(j, b):
        vb = vbase + j * CHUNK
        return pltpu.make_async_copy(fbufs[b],
                                     outf_hbm.at[pl.ds(vb * K, ROWS)],
                                     sems_w[b])

    def _wbq_desc(j, qb):
        vb = vbase + j * CHUNK
        return pltpu.make_async_copy(qbufs[qb],
                                     outq_hbm.at[pl.ds(vb, CHUNK)],
                                     sems_q[qb])

    def _gather_desc(j, b):
        return pltpu.make_async_copy(feat_hbm.at[idx_all.at[j]], fbufs[b],
                                     sems_g[b])

    def body(p, carry):
        for b in range(NBUF):
            j = NBUF * p + b

            @pl.when(p > 0)
            def _():
                _wbf_desc(j, b).wait()
                _wbq_desc(j, b).wait()

            _gather_desc(j, b).start()
        for b in range(NBUF):
            j = NBUF * p + b
            _qrows(j, qbufs[b])
            _gather_desc(j, b).wait()
            _wbf_desc(j, b).start()
            _wbq_desc(j, b).start()
        return carry

    nfull = NCH // NBUF
    lax.fori_loop(0, nfull, body, 0)
    for j in range(nfull * NBUF, NCH):
        b = j % NBUF
        _wbf_desc(j, b).wait()
        _wbq_desc(j, b).wait()
        _gather_desc(j, b).start()
        _qrows(j, qbufs[b])
        _gather_desc(j, b).wait()
        _wbf_desc(j, b).start()
        _wbq_desc(j, b).start()
    for j in range(NCH - NBUF, NCH):
        b = j % NBUF
        _wbf_desc(j, b).wait()
        _wbq_desc(j, b).wait()


def _make_sc_gather():
    mesh = plsc.VectorSubcoreMesh(core_axis_name="c", subcore_axis_name="s",
                                  num_cores=2, num_subcores=16)
    return pl.kernel(
        _sc_gather_body,
        mesh=mesh,
        out_type=[
            jax.ShapeDtypeStruct((VPAD * K, F), jnp.float32),
            jax.ShapeDtypeStruct((VPAD, 16, K), jnp.float32),
        ],
        scratch_types=(
            [
                pltpu.VMEM((NCH, ROWS), jnp.int32),
                pltpu.VMEM((NCH, ROWS), jnp.float32),
                pltpu.VMEM((C * V,), jnp.float32),
            ]
            + [pltpu.VMEM((ROWS, F), jnp.float32) for _ in range(NBUF)]
            + [pltpu.VMEM((CHUNK, 16, K), jnp.float32) for _ in range(NBUF)]
            + [pltpu.SemaphoreType.DMA for _ in range(3 * NBUF)]
        ),
        compiler_params=pltpu.CompilerParams(needs_layout_passes=False),
    )


def _prep_weights(W0, b0, W1, b1, W2, b2):
    w0e = jnp.zeros((HE, 16), jnp.float32)
    w0e = w0e.at[:H, :C * C].set(W0.T).at[:H, 9].set(b0).at[H, 9].set(1.0)
    w1e = jnp.zeros((HE, HE), jnp.float32)
    w1e = w1e.at[:H, :H].set(W1.T).at[:H, H].set(b1).at[H, H].set(1.0)
    w2e = jnp.zeros((C * C, HE), jnp.float32)
    w2e = w2e.at[:, :H].set(W2.T).at[:, H].set(b2)
    return w0e, w1e, w2e


def _tc_body(qt_ref, nf_ref, w0_ref, w1_ref, w2_ref, out_ref):
    qt = qt_ref[...]
    nf = nf_ref[...].reshape(VB, K, F)
    m = lax.dot_general(qt, nf, (((2,), (1,)), ((0,), (0,))),
                        preferred_element_type=jnp.float32)
    m2 = jnp.concatenate([m[v] for v in range(VB)], axis=1)
    recip = 1.0 / (m2[12:13, :] + 1e-4)
    mean = m2[9:12, :] * recip
    exx = m2[0:9, :] * recip
    crows = [exx[3 * c + d:3 * c + d + 1] - mean[c:c + 1] * mean[d:d + 1]
             for c in range(C) for d in range(C)]
    crows.append(jnp.ones((1, VB * F), jnp.float32))
    crows.append(jnp.zeros((6, VB * F), jnp.float32))
    cov2 = jnp.concatenate(crows, axis=0)
    x = lax.dot_general(w0_ref[...], cov2, (((1,), (0,)), ((), ())),
                        preferred_element_type=jnp.float32)
    x = jnp.where(x > 0, x, jnp.exp(x) - 1.0)
    x = lax.dot_general(w1_ref[...], x, (((1,), (0,)), ((), ())),
                        preferred_element_type=jnp.float32)
    x = jnp.where(x > 0, x, jnp.exp(x) - 1.0)
    x3 = lax.dot_general(w2_ref[...], x, (((1,), (0,)), ((), ())),
                         preferred_element_type=jnp.float32)
    r3 = x3.reshape(C * C, VB, F)
    out_ref[...] = jnp.transpose(r3, (1, 2, 0)).reshape(VB, F * C * C)


def _make_tc_call():
    return pl.pallas_call(
        _tc_body,
        grid=(V // VB,),
        in_specs=[
            pl.BlockSpec((VB, 16, K), lambda i: (i, 0, 0)),
            pl.BlockSpec((VB * K, F), lambda i: (i, 0)),
            pl.BlockSpec((HE, 16), lambda i: (0, 0)),
            pl.BlockSpec((HE, HE), lambda i: (0, 0)),
            pl.BlockSpec((C * C, HE), lambda i: (0, 0)),
        ],
        out_specs=pl.BlockSpec((VB, F * C * C), lambda i: (i, 0)),
        out_shape=jax.ShapeDtypeStruct((V, F * C * C), jnp.float32),
    )


_tc_call = _make_tc_call()


def kernel(coordinates, distsq, features, n_idxs, W0, b0, W1, b1, W2, b2):
    nidx = n_idxs.astype(jnp.int32)
    nidx_pad = jnp.zeros((VPAD, K), jnp.int32).at[:V].set(nidx)
    nidx_pad = nidx_pad.reshape(VPAD * K // ROWS, ROWS)
    dsq_pad = jnp.zeros((VPAD, K), jnp.float32).at[:V].set(distsq)
    dsq_pad = dsq_pad.reshape(VPAD * K // ROWS, ROWS)
    coordt = coordinates.T.reshape(C * V)
    nf_g, qt_g = _make_sc_gather()(nidx_pad, dsq_pad, features, coordt)
    w0e, w1e, w2e = _prep_weights(W0, b0, W1, b1, W2, b2)
    return _tc_call(qt_g, nf_g, w0e, w1e, w2e)
